# baseline (speedup 1.0000x reference)
import functools

import jax
import jax.numpy as jnp
from jax.experimental import pallas as pl
from jax.experimental.pallas import tpu as pltpu

NUM_NEURONS = 8192
INPUT_SIZE = 256
CHUNK_ROWS = 4096
NUM_CHUNKS = NUM_NEURONS // CHUNK_ROWS


def _feast_kernel(x_ref, w_ref, out_ref, *rest):
    bufs = rest[:NUM_CHUNKS]
    sems = rest[NUM_CHUNKS:]

    def copy(c):
        return pltpu.make_async_copy(
            w_ref.at[pl.ds(c * CHUNK_ROWS, CHUNK_ROWS), :], bufs[c], sems[c])

    for c in range(NUM_CHUNKS):
        copy(c).start()

    best_val = jnp.float32(jnp.inf)
    best_idx = jnp.int32(0)
    xv = x_ref[...]
    for c in range(NUM_CHUNKS):
        copy(c).wait()
        d = bufs[c][...] - xv
        vals = jnp.sum(d * d, axis=1, keepdims=True)
        m = jnp.min(vals)
        a = jnp.argmin(vals[:, 0]).astype(jnp.int32) + c * CHUNK_ROWS
        better = m < best_val
        best_val = jnp.where(better, m, best_val)
        best_idx = jnp.where(better, a, best_idx)

    flat_iota = (
        jax.lax.broadcasted_iota(jnp.int32, (64, 128), 0) * 128
        + jax.lax.broadcasted_iota(jnp.int32, (64, 128), 1)
    )
    out_ref[...] = (flat_iota == best_idx).astype(jnp.float32)


@functools.partial(jax.jit, static_argnames=("interpret",))
def kernel(x, weights, thresholds, interpret=False):
    del thresholds
    out = pl.pallas_call(
        _feast_kernel,
        in_specs=[
            pl.BlockSpec((1, INPUT_SIZE), lambda: (0, 0)),
            pl.BlockSpec(memory_space=pl.ANY),
        ],
        out_specs=pl.BlockSpec((64, 128), lambda: (0, 0)),
        out_shape=jax.ShapeDtypeStruct((64, 128), jnp.float32),
        scratch_shapes=(
            [pltpu.VMEM((CHUNK_ROWS, INPUT_SIZE), jnp.float32)
             for _ in range(NUM_CHUNKS)]
            + [pltpu.SemaphoreType.DMA for _ in range(NUM_CHUNKS)]
        ),
        interpret=interpret,
    )(x.reshape(1, INPUT_SIZE), weights)
    return out.reshape(NUM_NEURONS)


# speedup vs baseline: 1.1357x; 1.1357x over previous
import functools

import jax
import jax.numpy as jnp
from jax.experimental import pallas as pl
from jax.experimental.pallas import tpu as pltpu

NUM_NEURONS = 8192
INPUT_SIZE = 256
BLOCK_ROWS = 4096
NUM_BLOCKS = NUM_NEURONS // BLOCK_ROWS


def _feast_kernel(x_ref, w_ref, out_ref, min_ref, arg_ref):
    i = pl.program_id(0)

    d = w_ref[...] - x_ref[...]
    vals = jnp.sum(d * d, axis=1, keepdims=True)

    blk_min = jnp.min(vals)
    blk_arg = jnp.argmin(vals[:, 0]).astype(jnp.int32) + i * BLOCK_ROWS

    @pl.when(i == 0)
    def _init():
        min_ref[0] = blk_min
        arg_ref[0] = blk_arg

    @pl.when(i > 0)
    def _update():
        better = blk_min < min_ref[0]
        min_ref[0] = jnp.where(better, blk_min, min_ref[0])
        arg_ref[0] = jnp.where(better, blk_arg, arg_ref[0])

    @pl.when(i == NUM_BLOCKS - 1)
    def _finalize():
        idx = arg_ref[0]
        flat_iota = (
            jax.lax.broadcasted_iota(jnp.int32, (64, 128), 0) * 128
            + jax.lax.broadcasted_iota(jnp.int32, (64, 128), 1)
        )
        out_ref[...] = (flat_iota == idx).astype(jnp.float32)


@functools.partial(jax.jit, static_argnames=("interpret",))
def kernel(x, weights, thresholds, interpret=False):
    del thresholds
    out = pl.pallas_call(
        _feast_kernel,
        grid=(NUM_BLOCKS,),
        in_specs=[
            pl.BlockSpec((1, INPUT_SIZE), lambda i: (0, 0)),
            pl.BlockSpec((BLOCK_ROWS, INPUT_SIZE), lambda i: (i, 0)),
        ],
        out_specs=pl.BlockSpec((64, 128), lambda i: (0, 0)),
        out_shape=jax.ShapeDtypeStruct((64, 128), jnp.float32),
        scratch_shapes=[
            pltpu.SMEM((1,), jnp.float32),
            pltpu.SMEM((1,), jnp.int32),
        ],
        interpret=interpret,
    )(x.reshape(1, INPUT_SIZE), weights)
    return out.reshape(NUM_NEURONS)


# speedup vs baseline: 1.1429x; 1.0063x over previous
import functools

import jax
import jax.numpy as jnp
from jax.experimental import pallas as pl
from jax.experimental.pallas import tpu as pltpu

NUM_NEURONS = 8192
INPUT_SIZE = 256
BLOCK_ROWS = 4096
NUM_BLOCKS = NUM_NEURONS // BLOCK_ROWS


def _feast_kernel(x_ref, w_ref, out_ref, min_ref, arg_ref):
    i = pl.program_id(0)

    d = w_ref[...] - x_ref[...]
    vals = jnp.sum(d * d, axis=1)

    blk_min = jnp.min(vals)
    blk_arg = jnp.argmin(vals).astype(jnp.int32) + i * BLOCK_ROWS

    @pl.when(i == 0)
    def _init():
        min_ref[0] = blk_min
        arg_ref[0] = blk_arg

    @pl.when(i > 0)
    def _update():
        better = blk_min < min_ref[0]
        min_ref[0] = jnp.where(better, blk_min, min_ref[0])
        arg_ref[0] = jnp.where(better, blk_arg, arg_ref[0])

    @pl.when(i == NUM_BLOCKS - 1)
    def _finalize():
        idx = arg_ref[0]
        flat_iota = (
            jax.lax.broadcasted_iota(jnp.int32, (64, 128), 0) * 128
            + jax.lax.broadcasted_iota(jnp.int32, (64, 128), 1)
        )
        out_ref[...] = (flat_iota == idx).astype(jnp.float32)


@functools.partial(jax.jit, static_argnames=("interpret",))
def kernel(x, weights, thresholds, interpret=False):
    del thresholds
    out = pl.pallas_call(
        _feast_kernel,
        grid=(NUM_BLOCKS,),
        in_specs=[
            pl.BlockSpec((1, INPUT_SIZE), lambda i: (0, 0)),
            pl.BlockSpec((BLOCK_ROWS, INPUT_SIZE), lambda i: (i, 0)),
        ],
        out_specs=pl.BlockSpec((64, 128), lambda i: (0, 0)),
        out_shape=jax.ShapeDtypeStruct((64, 128), jnp.float32),
        scratch_shapes=[
            pltpu.SMEM((1,), jnp.float32),
            pltpu.SMEM((1,), jnp.int32),
        ],
        interpret=interpret,
    )(x.reshape(1, INPUT_SIZE), weights)
    return out.reshape(NUM_NEURONS)
